# Initial kernel scaffold; baseline (speedup 1.0000x reference)
import jax
import jax.numpy as jnp
from jax.experimental import pallas as pl


def kernel(x, edge_index, edge_attr, self_loop_index, self_loop_type, node_emb, We0, be0, W1_0, b1_0, gamma0, beta0, W2_0, b2_0, We1, be1, W1_1, b1_1, gamma1, beta1, W2_1, b2_1):
    raise NotImplementedError("write your pallas kernel here")


# speedup vs baseline: 9.3173x; 9.3173x over previous
import functools

import jax
import jax.numpy as jnp
from jax import lax
from jax.experimental import pallas as pl
from jax.experimental.pallas import tpu as pltpu
from jax.experimental.pallas import tpu_sc as plsc

_NC = 2
_NS = 16
_CHUNK = 128


def _phase_a_call(src, dst, ea_pad, tbl, zeros16, n_nodes, n_edges, edim):
    n_chunks = n_edges // _CHUNK
    n_workers = _NC * _NS
    cb, crem = n_chunks // n_workers, n_chunks % n_workers
    mesh = plsc.VectorSubcoreMesh(core_axis_name="c", subcore_axis_name="s")

    @functools.partial(
        pl.kernel,
        mesh=mesh,
        out_type=[jax.ShapeDtypeStruct((n_nodes, 16), jnp.float32)] * 2,
        compiler_params=pltpu.CompilerParams(use_tc_tiling_on_sc=False),
        scratch_types=[
            pltpu.VMEM((_CHUNK,), jnp.int32),
            pltpu.VMEM((_CHUNK,), jnp.int32),
            pltpu.VMEM((_CHUNK, 16), jnp.float32),
            pltpu.VMEM((_CHUNK, 16), jnp.float32),
            pltpu.VMEM_SHARED((n_nodes, 16), jnp.float32),
            pltpu.VMEM_SHARED((n_nodes, 16), jnp.float32),
            pltpu.SemaphoreType.DMA,
        ],
    )
    def k(src_h, dst_h, ea_h, tbl_h, z_h, out0, out1, srcb, dstb, rows, rows2,
          accum, tbl_s, sem):
        c = lax.axis_index("c")
        s = lax.axis_index("s")
        gwid = s * _NC + c

        @pl.when(s == 0)
        def _zero():
            pltpu.sync_copy(z_h, accum)

        @pl.when(s == 1)
        def _stage():
            pltpu.sync_copy(tbl_h, tbl_s)

        plsc.subcore_barrier()

        def chunk(i, carry):
            base = (i * n_workers + gwid) * _CHUNK
            pltpu.sync_copy(src_h.at[pl.ds(base, _CHUNK)], srcb)
            pltpu.sync_copy(dst_h.at[pl.ds(base, _CHUNK)], dstb)
            pltpu.sync_copy(ea_h.at[pl.ds(base, _CHUNK)], rows)
            pltpu.async_copy(tbl_s.at[srcb], rows2, sem).wait()
            pltpu.sync_copy(rows, accum.at[dstb], add=True)
            pltpu.sync_copy(rows2, accum.at[dstb], add=True)
            return carry

        lax.fori_loop(0, cb + (gwid < crem).astype(jnp.int32), chunk, 0)
        plsc.subcore_barrier()

        @pl.when(jnp.logical_and(s == 0, c == 0))
        def _out0():
            pltpu.sync_copy(accum, out0)

        @pl.when(jnp.logical_and(s == 0, c == 1))
        def _out1():
            pltpu.sync_copy(accum, out1)

    return k(src, dst, ea_pad, tbl, zeros16)


def _phase_b_call(src, dst, h, zeros_h, n_nodes, n_edges, hdim):
    n_chunks = n_edges // _CHUNK
    n_workers = _NC * _NS
    cb, crem = n_chunks // n_workers, n_chunks % n_workers
    mesh = plsc.VectorSubcoreMesh(core_axis_name="c", subcore_axis_name="s")

    @functools.partial(
        pl.kernel,
        mesh=mesh,
        out_type=[jax.ShapeDtypeStruct((n_nodes, hdim), jnp.float32)] * 2,
        scratch_types=[
            pltpu.VMEM((_CHUNK,), jnp.int32),
            pltpu.VMEM((_CHUNK,), jnp.int32),
            pltpu.VMEM((_CHUNK, hdim), jnp.float32),
            pltpu.VMEM_SHARED((n_nodes, hdim), jnp.float32),
            pltpu.SemaphoreType.DMA,
        ],
    )
    def k(src_h, dst_h, h_h, z_h, out0, out1, srcb, dstb, rows, accum, sem):
        c = lax.axis_index("c")
        s = lax.axis_index("s")
        gwid = s * _NC + c

        @pl.when(s == 0)
        def _zero():
            pltpu.sync_copy(z_h, accum)

        plsc.subcore_barrier()

        def chunk(i, carry):
            base = (i * n_workers + gwid) * _CHUNK
            pltpu.sync_copy(src_h.at[pl.ds(base, _CHUNK)], srcb)
            pltpu.sync_copy(dst_h.at[pl.ds(base, _CHUNK)], dstb)
            pltpu.async_copy(h_h.at[srcb], rows, sem).wait()
            pltpu.sync_copy(rows, accum.at[dstb], add=True)
            return carry

        lax.fori_loop(0, cb + (gwid < crem).astype(jnp.int32), chunk, 0)
        plsc.subcore_barrier()

        @pl.when(jnp.logical_and(s == 0, c == 0))
        def _out0():
            pltpu.sync_copy(accum, out0)

        @pl.when(jnp.logical_and(s == 0, c == 1))
        def _out1():
            pltpu.sync_copy(accum, out1)

    return k(src, dst, h, zeros_h)


_ROWS = 1000


def _front0_body(edim, hdim, pa0, pa1, xc, slr, emb, we, be, w1, b1,
                 z_ref, sadj_ref, st_ref):
    i = pl.program_id(0)
    s_adj = pa0[...] + pa1[...] + slr[...]
    sadj_ref[...] = s_adj
    w1a = w1[0:hdim, :]
    w1b = w1[hdim : 2 * hdim, :]
    memb = jnp.dot(emb[...], w1a, preferred_element_type=jnp.float32)
    q_ea = jnp.dot(we[...], w1b, preferred_element_type=jnp.float32)
    q_cnt = memb[1:2, :] - memb[0:1, :]
    q_deg = memb[0:1, :] + jnp.dot(be[...], w1b, preferred_element_type=jnp.float32)
    pad = jnp.zeros((16 - edim - 2, w1.shape[1]), jnp.float32)
    q = jnp.concatenate([q_ea, q_cnt, q_deg, pad], axis=0)
    z = jnp.dot(s_adj, q, preferred_element_type=jnp.float32)
    z = z + xc[...] * q_cnt + b1[...]
    z_ref[...] = z

    @pl.when(i == 0)
    def _():
        st_ref[...] = jnp.zeros_like(st_ref)

    st_ref[0:1, :] += jnp.sum(z, axis=0, keepdims=True)
    st_ref[1:2, :] += jnp.sum(z * z, axis=0, keepdims=True)


def _front1_body(edim, hdim, pb0, pb1, hblk, sadj, we, be, w1, b1,
                 z_ref, st_ref):
    i = pl.program_id(0)
    sh = pb0[...] + pb1[...] + hblk[...]
    w1a = w1[0:hdim, :]
    w1b = w1[hdim : 2 * hdim, :]
    t_ea = jnp.dot(we[...], w1b, preferred_element_type=jnp.float32)
    t_deg = jnp.dot(be[...], w1b, preferred_element_type=jnp.float32)
    zrow = jnp.zeros((1, w1.shape[1]), jnp.float32)
    pad = jnp.zeros((16 - edim - 2, w1.shape[1]), jnp.float32)
    t = jnp.concatenate([t_ea, zrow, t_deg, pad], axis=0)
    z = jnp.dot(sh, w1a, preferred_element_type=jnp.float32)
    z = z + jnp.dot(sadj[...], t, preferred_element_type=jnp.float32) + b1[...]
    z_ref[...] = z

    @pl.when(i == 0)
    def _():
        st_ref[...] = jnp.zeros_like(st_ref)

    st_ref[0:1, :] += jnp.sum(z, axis=0, keepdims=True)
    st_ref[1:2, :] += jnp.sum(z * z, axis=0, keepdims=True)


def _back_body(n_nodes, final_relu, z, st, gamma, beta, w2, b2, out_ref):
    inv_n = 1.0 / n_nodes
    mean = st[0:1, :] * inv_n
    var = st[1:2, :] * inv_n - mean * mean
    inv = lax.rsqrt(var + 1e-5)
    zn = (z[...] - mean) * inv * gamma[...] + beta[...]
    a = jnp.maximum(zn, 0.0)
    o = jnp.dot(a, w2[...], preferred_element_type=jnp.float32) + b2[...]
    if final_relu:
        o = jnp.maximum(o, 0.0)
    out_ref[...] = o


def _tc_front0(pa0, pa1, x_col, sl_row, emb, we, be, w1, b1, n_nodes, edim, hdim):
    grid = (n_nodes // _ROWS,)
    h2 = 2 * hdim
    blk = lambda r, k: pl.BlockSpec((r, k), lambda b: (b, 0))
    fix = lambda r, k: pl.BlockSpec((r, k), lambda b: (0, 0))
    return pl.pallas_call(
        functools.partial(_front0_body, edim, hdim),
        grid=grid,
        in_specs=[
            blk(_ROWS, 16), blk(_ROWS, 16), blk(_ROWS, 1), fix(1, 16),
            fix(2, hdim), fix(edim, hdim), fix(1, hdim), fix(h2, h2), fix(1, h2),
        ],
        out_specs=[blk(_ROWS, h2), blk(_ROWS, 16), fix(8, h2)],
        out_shape=[
            jax.ShapeDtypeStruct((n_nodes, h2), jnp.float32),
            jax.ShapeDtypeStruct((n_nodes, 16), jnp.float32),
            jax.ShapeDtypeStruct((8, h2), jnp.float32),
        ],
    )(pa0, pa1, x_col, sl_row, emb, we, be, w1, b1)


def _tc_front1(pb0, pb1, h, sadj, we, be, w1, b1, n_nodes, edim, hdim):
    grid = (n_nodes // _ROWS,)
    h2 = 2 * hdim
    blk = lambda r, k: pl.BlockSpec((r, k), lambda b: (b, 0))
    fix = lambda r, k: pl.BlockSpec((r, k), lambda b: (0, 0))
    return pl.pallas_call(
        functools.partial(_front1_body, edim, hdim),
        grid=grid,
        in_specs=[
            blk(_ROWS, hdim), blk(_ROWS, hdim), blk(_ROWS, hdim), blk(_ROWS, 16),
            fix(edim, hdim), fix(1, hdim), fix(h2, h2), fix(1, h2),
        ],
        out_specs=[blk(_ROWS, h2), fix(8, h2)],
        out_shape=[
            jax.ShapeDtypeStruct((n_nodes, h2), jnp.float32),
            jax.ShapeDtypeStruct((8, h2), jnp.float32),
        ],
    )(pb0, pb1, h, sadj, we, be, w1, b1)


def _tc_back(z, st, gamma, beta, w2, b2, n_nodes, hdim, final_relu):
    grid = (n_nodes // _ROWS,)
    h2 = 2 * hdim
    blk = lambda r, k: pl.BlockSpec((r, k), lambda b: (b, 0))
    fix = lambda r, k: pl.BlockSpec((r, k), lambda b: (0, 0))
    return pl.pallas_call(
        functools.partial(_back_body, n_nodes, final_relu),
        grid=grid,
        in_specs=[
            blk(_ROWS, h2), fix(8, h2), fix(1, h2), fix(1, h2),
            fix(h2, hdim), fix(1, hdim),
        ],
        out_specs=pl.BlockSpec((_ROWS, hdim), lambda b: (b, 0)),
        out_shape=jax.ShapeDtypeStruct((n_nodes, hdim), jnp.float32),
    )(z, st, gamma, beta, w2, b2)


def kernel(x, edge_index, edge_attr, self_loop_index, self_loop_type, node_emb,
           We0, be0, W1_0, b1_0, gamma0, beta0, W2_0, b2_0,
           We1, be1, W1_1, b1_1, gamma1, beta1, W2_1, b2_1):
    n_nodes = x.shape[0]
    n_edges, edim = edge_attr.shape
    hdim = node_emb.shape[1]

    dst = edge_index[0].astype(jnp.int32)
    src = edge_index[1].astype(jnp.int32)
    x_f = x.astype(jnp.float32)

    ea_pad = jnp.concatenate(
        [
            edge_attr.astype(jnp.float32),
            jnp.zeros((n_edges, 16 - edim), jnp.float32),
        ],
        axis=1,
    )
    tbl = jnp.zeros((n_nodes, 16), jnp.float32)
    tbl = tbl.at[:, edim].set(x_f).at[:, edim + 1].set(1.0)
    slt_f = jnp.asarray(self_loop_type, jnp.float32)
    sl_row = jnp.zeros((1, 16), jnp.float32).at[0, self_loop_index].add(slt_f)
    sl_row = sl_row.at[0, edim + 1].add(1.0)

    zeros16 = jnp.zeros((n_nodes, 16), jnp.float32)
    zeros_h = jnp.zeros((n_nodes, hdim), jnp.float32)

    pa0, pa1 = _phase_a_call(src, dst, ea_pad, tbl, zeros16, n_nodes, n_edges, edim)

    x_col = x_f.reshape(n_nodes, 1)
    emb = node_emb.astype(jnp.float32)
    z0, sadj, st0 = _tc_front0(
        pa0, pa1, x_col, sl_row, emb, We0, be0.reshape(1, hdim),
        W1_0, b1_0.reshape(1, 2 * hdim), n_nodes, edim, hdim)
    h = _tc_back(z0, st0, gamma0.reshape(1, -1), beta0.reshape(1, -1),
                 W2_0, b2_0.reshape(1, hdim), n_nodes, hdim, final_relu=True)

    pb0, pb1 = _phase_b_call(src, dst, h, zeros_h, n_nodes, n_edges, hdim)

    z1, st1 = _tc_front1(
        pb0, pb1, h, sadj, We1, be1.reshape(1, hdim),
        W1_1, b1_1.reshape(1, 2 * hdim), n_nodes, edim, hdim)
    out = _tc_back(z1, st1, gamma1.reshape(1, -1), beta1.reshape(1, -1),
                   W2_1, b2_1.reshape(1, hdim), n_nodes, hdim, final_relu=False)
    return out
